# baseline (speedup 1.0000x reference)
import jax
import jax.numpy as jnp
from jax.experimental import pallas as pl
from jax.experimental.pallas import tpu as pltpu

N = 8192
E = 8192
BM = 256
BM3 = 512
_EPS = 1e-5


def _dot(a, b):
    return jax.lax.dot_general(
        a, b, (((1,), (0,)), ((), ())),
        precision=jax.lax.Precision.HIGHEST,
        preferred_element_type=jnp.float32)


def _bn_relu_max(zc, g, be):
    m = jnp.mean(zc, axis=0, keepdims=True)
    v = jnp.mean(jnp.square(zc), axis=0, keepdims=True) - jnp.square(m)
    zp = jax.nn.relu((zc - m) * jax.lax.rsqrt(v + _EPS) * g + be)
    return jnp.max(zp, axis=1, keepdims=True)


def _prep1_body(xn_ref, xe_ref, w1t_ref, hw1t_ref, hb1_ref, comb_ref):
    comb_ref[:, :32] = _dot(xn_ref[:], w1t_ref[:])
    comb_ref[:, 32:40] = _dot(xe_ref[:], hw1t_ref[:]) + hb1_ref[:]
    comb_ref[:, 40:41] = jnp.ones((E, 1), jnp.float32)


def _stream1_body(comb_ref, b1_ref, a_ref, l1_ref, y1_ref, zca_ref):
    y1_ref[:] = jax.nn.relu(_dot(a_ref[:], comb_ref[:, :32]) + b1_ref[:])
    zca_ref[:] = _dot(l1_ref[:], comb_ref[:, 32:41])


def _prep2_body(y1_ref, zca_ref, w2t_ref, g1_ref, be1_ref, comb_ref):
    comb_ref[:, :32] = _dot(y1_ref[:], w2t_ref[:])
    comb_ref[:, 32:33] = _bn_relu_max(zca_ref[:, :8], g1_ref[:], be1_ref[:])


def _stream2_body(comb_ref, b2_ref, a_ref, l1_ref, h_ref, u_ref):
    h_ref[:] = jax.nn.relu(_dot(a_ref[:], comb_ref[:, :32]) + b2_ref[:])
    u_ref[:] = _dot(l1_ref[:], comb_ref[:, 32:33])


def _prep3_body(u_ref, zca_ref, comb2_ref, hw2t_ref, hb2_ref, g2_ref,
                be2_ref, ehwt_ref, ehb_ref, zh_ref, ep_ref):
    zc2 = u_ref[:] * hw2t_ref[:] + zca_ref[:, 8:9] * hb2_ref[:]
    z2 = _bn_relu_max(zc2, g2_ref[:], be2_ref[:])
    zh_ref[:, 0:1] = comb2_ref[:, 32:33]
    zh_ref[:, 1:2] = z2
    ep_ref[:] = jax.nn.sigmoid(_dot(zh_ref[:], ehwt_ref[:]) + ehb_ref[:])


def _stream3_body(zh_ref, nhwt_ref, nhb_ref, b1m_ref, h_ref,
                  hcat_ref, np_ref):
    hcat_ref[:, :32] = h_ref[:]
    hcat_ref[:, 32:34] = _dot(b1m_ref[:], zh_ref[:])
    np_ref[:] = jax.nn.sigmoid(_dot(hcat_ref[:], nhwt_ref[:]) + nhb_ref[:])


def _full(shape):
    return pl.BlockSpec(shape, lambda *_: (0,) * len(shape))


def _rows(width, bm=BM):
    return pl.BlockSpec((bm, width), lambda i: (i, 0))


def kernel(X_n, X_e, A_tilde, L1_tilde, B1, gnn_w1, gnn_b1, gnn_w2, gnn_b2,
           hosc1_w, hosc1_b, hosc1_g, hosc1_be, hosc2_w, hosc2_b, hosc2_g,
           hosc2_be, nh_w, nh_b, eh_w, eh_b):
    f32 = jnp.float32
    grid = (N // BM,)

    comb1 = pl.pallas_call(
        _prep1_body,
        in_specs=[_full((N, 128)), _full((E, 16)), _full((128, 32)),
                  _full((16, 8)), _full((1, 8))],
        out_specs=_full((N, 41)),
        out_shape=jax.ShapeDtypeStruct((N, 41), f32),
    )(X_n, X_e, gnn_w1.T, hosc1_w.T, hosc1_b.reshape(1, -1))

    y1, zca = pl.pallas_call(
        _stream1_body,
        grid=grid,
        in_specs=[_full((N, 41)), _full((1, 32)), _rows(N), _rows(E)],
        out_specs=[_rows(32), _rows(9)],
        out_shape=[jax.ShapeDtypeStruct((N, 32), f32),
                   jax.ShapeDtypeStruct((E, 9), f32)],
    )(comb1, gnn_b1.reshape(1, -1), A_tilde, L1_tilde)

    comb2 = pl.pallas_call(
        _prep2_body,
        in_specs=[_full((N, 32)), _full((E, 9)), _full((32, 32)),
                  _full((1, 8)), _full((1, 8))],
        out_specs=_full((N, 33)),
        out_shape=jax.ShapeDtypeStruct((N, 33), f32),
    )(y1, zca, gnn_w2.T, hosc1_g.reshape(1, -1), hosc1_be.reshape(1, -1))

    h, u = pl.pallas_call(
        _stream2_body,
        grid=grid,
        in_specs=[_full((N, 33)), _full((1, 32)), _rows(N), _rows(E)],
        out_specs=[_rows(32), _rows(1)],
        out_shape=[jax.ShapeDtypeStruct((N, 32), f32),
                   jax.ShapeDtypeStruct((E, 1), f32)],
    )(comb2, gnn_b2.reshape(1, -1), A_tilde, L1_tilde)

    zh, ep = pl.pallas_call(
        _prep3_body,
        in_specs=[_full((E, 1)), _full((E, 9)), _full((N, 33)),
                  _full((1, 8)), _full((1, 8)), _full((1, 8)), _full((1, 8)),
                  _full((2, 1)), _full((1, 1))],
        out_specs=[_full((E, 2)), _full((E, 1))],
        out_shape=[jax.ShapeDtypeStruct((E, 2), f32),
                   jax.ShapeDtypeStruct((E, 1), f32)],
    )(u, zca, comb2, hosc2_w.T, hosc2_b.reshape(1, -1),
      hosc2_g.reshape(1, -1), hosc2_be.reshape(1, -1), eh_w.T,
      eh_b.reshape(1, -1))

    hcat, np_ = pl.pallas_call(
        _stream3_body,
        grid=(N // BM3,),
        in_specs=[_full((E, 2)), _full((34, 1)), _full((1, 1)),
                  _rows(E, BM3), _rows(32, BM3)],
        out_specs=[_rows(34, BM3), _rows(1, BM3)],
        out_shape=[jax.ShapeDtypeStruct((N, 34), f32),
                   jax.ShapeDtypeStruct((N, 1), f32)],
    )(zh, nh_w.T, nh_b.reshape(1, -1), B1, h)

    return np_[:, 0], ep[:, 0], hcat


# speedup vs baseline: 2.5614x; 2.5614x over previous
import jax
import jax.numpy as jnp
from jax.experimental import pallas as pl
from jax.experimental.pallas import tpu as pltpu

N = 8192
E = 8192
BM = 256
BM3 = 512
_EPS = 1e-5


def _dot(a, b):
    return jax.lax.dot_general(
        a, b, (((1,), (0,)), ((), ())),
        precision=jax.lax.Precision.DEFAULT,
        preferred_element_type=jnp.float32)


def _bn_relu_max(zc, g, be):
    m = jnp.mean(zc, axis=0, keepdims=True)
    v = jnp.mean(jnp.square(zc), axis=0, keepdims=True) - jnp.square(m)
    zp = jax.nn.relu((zc - m) * jax.lax.rsqrt(v + _EPS) * g + be)
    return jnp.max(zp, axis=1, keepdims=True)


def _prep1_body(xn_ref, xe_ref, w1t_ref, hw1t_ref, hb1_ref, comb_ref):
    comb_ref[:, :32] = _dot(xn_ref[:], w1t_ref[:])
    comb_ref[:, 32:40] = _dot(xe_ref[:], hw1t_ref[:]) + hb1_ref[:]
    comb_ref[:, 40:41] = jnp.ones((E, 1), jnp.float32)


def _stream1_body(comb_ref, b1_ref, a_ref, l1_ref, y1_ref, zca_ref):
    y1_ref[:] = jax.nn.relu(_dot(a_ref[:], comb_ref[:, :32]) + b1_ref[:])
    zca_ref[:] = _dot(l1_ref[:], comb_ref[:, 32:41])


def _prep2_body(y1_ref, zca_ref, w2t_ref, g1_ref, be1_ref, comb_ref):
    comb_ref[:, :32] = _dot(y1_ref[:], w2t_ref[:])
    comb_ref[:, 32:33] = _bn_relu_max(zca_ref[:, :8], g1_ref[:], be1_ref[:])


def _stream2_body(comb_ref, b2_ref, a_ref, l1_ref, h_ref, u_ref):
    h_ref[:] = jax.nn.relu(_dot(a_ref[:], comb_ref[:, :32]) + b2_ref[:])
    u_ref[:] = _dot(l1_ref[:], comb_ref[:, 32:33])


def _prep3_body(u_ref, zca_ref, comb2_ref, hw2t_ref, hb2_ref, g2_ref,
                be2_ref, ehwt_ref, ehb_ref, zh_ref, ep_ref):
    zc2 = u_ref[:] * hw2t_ref[:] + zca_ref[:, 8:9] * hb2_ref[:]
    z2 = _bn_relu_max(zc2, g2_ref[:], be2_ref[:])
    zh_ref[:, 0:1] = comb2_ref[:, 32:33]
    zh_ref[:, 1:2] = z2
    ep_ref[:] = jax.nn.sigmoid(_dot(zh_ref[:], ehwt_ref[:]) + ehb_ref[:])


def _stream3_body(zh_ref, nhwt_ref, nhb_ref, b1m_ref, h_ref,
                  hcat_ref, np_ref):
    hcat_ref[:, :32] = h_ref[:]
    hcat_ref[:, 32:34] = _dot(b1m_ref[:], zh_ref[:])
    np_ref[:] = jax.nn.sigmoid(_dot(hcat_ref[:], nhwt_ref[:]) + nhb_ref[:])


def _full(shape):
    return pl.BlockSpec(shape, lambda *_: (0,) * len(shape))


def _rows(width, bm=BM):
    return pl.BlockSpec((bm, width), lambda i: (i, 0))


def kernel(X_n, X_e, A_tilde, L1_tilde, B1, gnn_w1, gnn_b1, gnn_w2, gnn_b2,
           hosc1_w, hosc1_b, hosc1_g, hosc1_be, hosc2_w, hosc2_b, hosc2_g,
           hosc2_be, nh_w, nh_b, eh_w, eh_b):
    f32 = jnp.float32
    grid = (N // BM,)

    comb1 = pl.pallas_call(
        _prep1_body,
        in_specs=[_full((N, 128)), _full((E, 16)), _full((128, 32)),
                  _full((16, 8)), _full((1, 8))],
        out_specs=_full((N, 41)),
        out_shape=jax.ShapeDtypeStruct((N, 41), f32),
    )(X_n, X_e, gnn_w1.T, hosc1_w.T, hosc1_b.reshape(1, -1))

    y1, zca = pl.pallas_call(
        _stream1_body,
        grid=grid,
        in_specs=[_full((N, 41)), _full((1, 32)), _rows(N), _rows(E)],
        out_specs=[_rows(32), _rows(9)],
        out_shape=[jax.ShapeDtypeStruct((N, 32), f32),
                   jax.ShapeDtypeStruct((E, 9), f32)],
    )(comb1, gnn_b1.reshape(1, -1), A_tilde, L1_tilde)

    comb2 = pl.pallas_call(
        _prep2_body,
        in_specs=[_full((N, 32)), _full((E, 9)), _full((32, 32)),
                  _full((1, 8)), _full((1, 8))],
        out_specs=_full((N, 33)),
        out_shape=jax.ShapeDtypeStruct((N, 33), f32),
    )(y1, zca, gnn_w2.T, hosc1_g.reshape(1, -1), hosc1_be.reshape(1, -1))

    h, u = pl.pallas_call(
        _stream2_body,
        grid=grid,
        in_specs=[_full((N, 33)), _full((1, 32)), _rows(N), _rows(E)],
        out_specs=[_rows(32), _rows(1)],
        out_shape=[jax.ShapeDtypeStruct((N, 32), f32),
                   jax.ShapeDtypeStruct((E, 1), f32)],
    )(comb2, gnn_b2.reshape(1, -1), A_tilde, L1_tilde)

    zh, ep = pl.pallas_call(
        _prep3_body,
        in_specs=[_full((E, 1)), _full((E, 9)), _full((N, 33)),
                  _full((1, 8)), _full((1, 8)), _full((1, 8)), _full((1, 8)),
                  _full((2, 1)), _full((1, 1))],
        out_specs=[_full((E, 2)), _full((E, 1))],
        out_shape=[jax.ShapeDtypeStruct((E, 2), f32),
                   jax.ShapeDtypeStruct((E, 1), f32)],
    )(u, zca, comb2, hosc2_w.T, hosc2_b.reshape(1, -1),
      hosc2_g.reshape(1, -1), hosc2_be.reshape(1, -1), eh_w.T,
      eh_b.reshape(1, -1))

    hcat, np_ = pl.pallas_call(
        _stream3_body,
        grid=(N // BM3,),
        in_specs=[_full((E, 2)), _full((34, 1)), _full((1, 1)),
                  _rows(E, BM3), _rows(32, BM3)],
        out_specs=[_rows(34, BM3), _rows(1, BM3)],
        out_shape=[jax.ShapeDtypeStruct((N, 34), f32),
                   jax.ShapeDtypeStruct((N, 1), f32)],
    )(zh, nh_w.T, nh_b.reshape(1, -1), B1, h)

    return np_[:, 0], ep[:, 0], hcat
